# baseline (speedup 1.0000x reference)
import functools

import jax
import jax.numpy as jnp
from jax import lax
from jax.experimental import pallas as pl
from jax.experimental.pallas import tpu as pltpu
from jax.experimental.pallas import tpu_sc as plsc

N = 10000
E = 320000
D = 128
LATENT = 64
NUM_GRAPHS = 64

NC = 2
NS = 16
NW = NC * NS
CH = 128
NCHUNK = 80
G = 40
NGROUP = NCHUNK // G
PER_W = CH * NCHUNK
N_TRASH = 240
N_PAD = N + N_TRASH
ROWS_PER_TILE = N_PAD // NS
NBUF = 2

def _dot(a, b):
    return jnp.dot(a, b, preferred_element_type=jnp.float32)


def _sc_scatter_body(src_hbm, dst_hbm, h_hbm, zeros_hbm, out_hbm,
                     src_v, dst_v, rows, sems, agg_sh):
    c = lax.axis_index("c")
    s = lax.axis_index("s")
    wid = c * NS + s

    HC = CH // 2

    def _start(j, b):
        pltpu.async_copy(h_hbm.at[src_v.at[j, pl.ds(0, HC)]],
                         rows[b].at[pl.ds(0, HC)], sems[b][0])
        pltpu.async_copy(h_hbm.at[src_v.at[j, pl.ds(HC, HC)]],
                         rows[b].at[pl.ds(HC, HC)], sems[b][1])

    def _wait(j, b):
        pltpu.make_async_copy(h_hbm.at[src_v.at[j, pl.ds(0, HC)]],
                              rows[b].at[pl.ds(0, HC)], sems[b][0]).wait()
        pltpu.make_async_copy(h_hbm.at[src_v.at[j, pl.ds(HC, HC)]],
                              rows[b].at[pl.ds(HC, HC)], sems[b][1]).wait()

    pltpu.sync_copy(src_hbm.at[wid, pl.ds(0, G)], src_v)
    pltpu.sync_copy(dst_hbm.at[wid, pl.ds(0, G)], dst_v)
    for b in range(NBUF):
        _start(b, b)

    pltpu.sync_copy(zeros_hbm, agg_sh.at[pl.ds(s * ROWS_PER_TILE, ROWS_PER_TILE)])

    plsc.subcore_barrier()

    for g in range(NGROUP):
        if g > 0:
            pltpu.sync_copy(src_hbm.at[wid, pl.ds(g * G, G)], src_v)
            pltpu.sync_copy(dst_hbm.at[wid, pl.ds(g * G, G)], dst_v)

            for b in range(NBUF):
                _start(b, b)

        @pl.loop(0, G // NBUF)
        def _chunk_loop(k):
            for b in range(NBUF):
                j = k * NBUF + b
                _wait(j, b)
                pltpu.sync_copy(rows[b], agg_sh.at[dst_v.at[j]], add=True)

                @pl.when(j + NBUF < G)
                def _():
                    _start(j + NBUF, b)

        for j in range(G - G % NBUF, G):
            b = j % NBUF
            _wait(j, b)
            pltpu.sync_copy(rows[b], agg_sh.at[dst_v.at[j]], add=True)

    plsc.subcore_barrier()

    pltpu.sync_copy(
        agg_sh.at[pl.ds(s * ROWS_PER_TILE, ROWS_PER_TILE)],
        out_hbm.at[c, pl.ds(s * ROWS_PER_TILE, ROWS_PER_TILE)],
    )


_sc_scatter = pl.kernel(
    _sc_scatter_body,
    out_type=jax.ShapeDtypeStruct((NC, N_PAD, D), jnp.float32),
    mesh=plsc.VectorSubcoreMesh(core_axis_name="c", subcore_axis_name="s"),
    scratch_types=dict(
        src_v=pltpu.VMEM((G, CH), jnp.int32),
        dst_v=pltpu.VMEM((G, CH), jnp.int32),
        rows=[pltpu.VMEM((CH, D), jnp.float32) for _ in range(NBUF)],
        sems=[[pltpu.SemaphoreType.DMA, pltpu.SemaphoreType.DMA]
              for _ in range(NBUF)],
        agg_sh=pltpu.VMEM_SHARED((N_PAD, D), jnp.float32),
    ),
)


BLK = 1000
GRID = N // BLK


def _tc_in_body(x_ref, w_ref, b_ref, o_ref):
    o_ref[...] = jnp.maximum(_dot(x_ref[...], w_ref[...]) + b_ref[...], 0.0)


def _tc_in(x, w, b):
    return pl.pallas_call(
        _tc_in_body,
        grid=(GRID,),
        in_specs=[
            pl.BlockSpec((BLK, D), lambda i: (i, 0)),
            pl.BlockSpec((D, D), lambda i: (0, 0)),
            pl.BlockSpec((1, D), lambda i: (0, 0)),
        ],
        out_specs=pl.BlockSpec((BLK, D), lambda i: (i, 0)),
        out_shape=jax.ShapeDtypeStruct((N, D), jnp.float32),
    )(x, w, b)


def _tc_conv_body(parts_ref, h_ref, w_ref, b_ref, o_ref):
    a = parts_ref[0] + parts_ref[1] + h_ref[...]
    o_ref[...] = jnp.maximum(_dot(a, w_ref[...]) + b_ref[...], 0.0)


def _tc_conv(parts, h, w, b):
    return pl.pallas_call(
        _tc_conv_body,
        grid=(GRID,),
        in_specs=[
            pl.BlockSpec((NC, BLK, D), lambda i: (0, i, 0)),
            pl.BlockSpec((BLK, D), lambda i: (i, 0)),
            pl.BlockSpec((D, D), lambda i: (0, 0)),
            pl.BlockSpec((1, D), lambda i: (0, 0)),
        ],
        out_specs=pl.BlockSpec((BLK, D), lambda i: (i, 0)),
        out_shape=jax.ShapeDtypeStruct((N, D), jnp.float32),
    )(parts, h, w, b)


def _tc_final_body(parts_ref, h_ref, w_ref, b_ref, batch_ref,
                   wm_ref, bm_ref, wl_ref, bl_ref,
                   zm_ref, zl_ref, ge_acc):
    i = pl.program_id(0)
    a = parts_ref[0] + parts_ref[1] + h_ref[...]
    h3 = jnp.maximum(_dot(a, w_ref[...]) + b_ref[...], 0.0)

    batch_row = batch_ref[0, 0, :]
    gids = lax.broadcasted_iota(jnp.int32, (NUM_GRAPHS, BLK), 0)
    onehot = (batch_row[None, :] == gids).astype(jnp.float32)

    @pl.when(i == 0)
    def _():
        ge_acc[...] = jnp.zeros_like(ge_acc)

    ge_acc[...] += _dot(onehot, h3)

    @pl.when(i == GRID - 1)
    def _():
        ge = ge_acc[...]
        zm_ref[...] = _dot(ge, wm_ref[...]) + bm_ref[...]
        zl_ref[...] = _dot(ge, wl_ref[...]) + bl_ref[...]


def _tc_final(parts, h, w, b, batch_r, wm, bm, wl, bl):
    return pl.pallas_call(
        _tc_final_body,
        grid=(GRID,),
        in_specs=[
            pl.BlockSpec((NC, BLK, D), lambda i: (0, i, 0)),
            pl.BlockSpec((BLK, D), lambda i: (i, 0)),
            pl.BlockSpec((D, D), lambda i: (0, 0)),
            pl.BlockSpec((1, D), lambda i: (0, 0)),
            pl.BlockSpec((1, 1, BLK), lambda i: (i, 0, 0)),
            pl.BlockSpec((D, LATENT), lambda i: (0, 0)),
            pl.BlockSpec((1, LATENT), lambda i: (0, 0)),
            pl.BlockSpec((D, LATENT), lambda i: (0, 0)),
            pl.BlockSpec((1, LATENT), lambda i: (0, 0)),
        ],
        out_specs=[
            pl.BlockSpec((NUM_GRAPHS, LATENT), lambda i: (0, 0)),
            pl.BlockSpec((NUM_GRAPHS, LATENT), lambda i: (0, 0)),
        ],
        out_shape=[
            jax.ShapeDtypeStruct((NUM_GRAPHS, LATENT), jnp.float32),
            jax.ShapeDtypeStruct((NUM_GRAPHS, LATENT), jnp.float32),
        ],
        scratch_shapes=[pltpu.VMEM((NUM_GRAPHS, D), jnp.float32)],
    )(parts, h, w, b, batch_r, wm, bm, wl, bl)


def kernel(x, edge_index, batch, W_in, b_in, W0, b0, W1, b1, W2, b2,
           W_mean, b_mean, W_logvar, b_logvar):
    src = edge_index[0].astype(jnp.int32)
    dst = edge_index[1].astype(jnp.int32)
    pad = NW * PER_W - E
    trash = N + jnp.arange(pad, dtype=jnp.int32) % N_TRASH
    pad_src = jnp.arange(pad, dtype=jnp.int32) % N
    src_p = jnp.concatenate([src, pad_src]).reshape(NW, PER_W)
    dst_p = jnp.concatenate([dst, trash]).reshape(NW, PER_W)
    src_r = src_p.reshape(NW, NCHUNK, CH)
    dst_r = dst_p.reshape(NW, NCHUNK, CH)
    zeros_hbm = jnp.zeros((ROWS_PER_TILE, D), jnp.float32)
    batch_r = batch.astype(jnp.int32).reshape(GRID, 1, BLK)

    b_in2 = b_in.reshape(1, D)
    b02, b12, b22 = b0.reshape(1, D), b1.reshape(1, D), b2.reshape(1, D)
    bm2 = b_mean.reshape(1, LATENT)
    bl2 = b_logvar.reshape(1, LATENT)

    h = _tc_in(x, W_in, b_in2)
    for w, bb in ((W0, b02), (W1, b12)):
        parts = _sc_scatter(src_r, dst_r, h, zeros_hbm)
        h = _tc_conv(parts, h, w, bb)
    parts = _sc_scatter(src_r, dst_r, h, zeros_hbm)
    z_mean, z_logvar = _tc_final(parts, h, W2, b22, batch_r,
                                 W_mean, bm2, W_logvar, bl2)
    return (z_mean, z_logvar)


# speedup vs baseline: 1.0387x; 1.0387x over previous
import functools

import jax
import jax.numpy as jnp
from jax import lax
from jax.experimental import pallas as pl
from jax.experimental.pallas import tpu as pltpu
from jax.experimental.pallas import tpu_sc as plsc

N = 10000
E = 320000
D = 128
LATENT = 64
NUM_GRAPHS = 64

NC = 2
NS = 16
NW = NC * NS
CH = 64
NCHUNK = 160
G = 32
NGROUP = NCHUNK // G
PER_W = CH * NCHUNK
N_TRASH = 112
N_PAD = N + N_TRASH
ROWS_PER_TILE = N_PAD // NS
NBUF = 4

def _dot(a, b):
    return jnp.dot(a, b, preferred_element_type=jnp.float32)


def _sc_scatter_body(src_hbm, dst_hbm, h_hbm, zeros_hbm, out_hbm,
                     src_v, dst_v, rows, sems, agg_sh):
    c = lax.axis_index("c")
    s = lax.axis_index("s")
    wid = c * NS + s

    def _start(j, b):
        pltpu.async_copy(h_hbm.at[src_v.at[j]], rows[b], sems[b])

    def _wait(j, b):
        pltpu.make_async_copy(h_hbm.at[src_v.at[j]], rows[b], sems[b]).wait()

    pltpu.sync_copy(src_hbm.at[wid, pl.ds(0, G)], src_v)
    pltpu.sync_copy(dst_hbm.at[wid, pl.ds(0, G)], dst_v)
    for b in range(NBUF):
        _start(b, b)

    pltpu.sync_copy(zeros_hbm, agg_sh.at[pl.ds(s * ROWS_PER_TILE, ROWS_PER_TILE)])

    plsc.subcore_barrier()

    for g in range(NGROUP):
        if g > 0:
            pltpu.sync_copy(src_hbm.at[wid, pl.ds(g * G, G)], src_v)
            pltpu.sync_copy(dst_hbm.at[wid, pl.ds(g * G, G)], dst_v)

            for b in range(NBUF):
                _start(b, b)

        @pl.loop(0, G // NBUF)
        def _chunk_loop(k):
            for b in range(NBUF):
                j = k * NBUF + b
                _wait(j, b)
                pltpu.sync_copy(rows[b], agg_sh.at[dst_v.at[j]], add=True)

                @pl.when(j + NBUF < G)
                def _():
                    _start(j + NBUF, b)

        for j in range(G - G % NBUF, G):
            b = j % NBUF
            _wait(j, b)
            pltpu.sync_copy(rows[b], agg_sh.at[dst_v.at[j]], add=True)

    plsc.subcore_barrier()

    pltpu.sync_copy(
        agg_sh.at[pl.ds(s * ROWS_PER_TILE, ROWS_PER_TILE)],
        out_hbm.at[c, pl.ds(s * ROWS_PER_TILE, ROWS_PER_TILE)],
    )


_sc_scatter = pl.kernel(
    _sc_scatter_body,
    out_type=jax.ShapeDtypeStruct((NC, N_PAD, D), jnp.float32),
    mesh=plsc.VectorSubcoreMesh(core_axis_name="c", subcore_axis_name="s"),
    scratch_types=dict(
        src_v=pltpu.VMEM((G, CH), jnp.int32),
        dst_v=pltpu.VMEM((G, CH), jnp.int32),
        rows=[pltpu.VMEM((CH, D), jnp.float32) for _ in range(NBUF)],
        sems=[pltpu.SemaphoreType.DMA for _ in range(NBUF)],
        agg_sh=pltpu.VMEM_SHARED((N_PAD, D), jnp.float32),
    ),
)


BLK = 1000
GRID = N // BLK


def _tc_in_body(x_ref, w_ref, b_ref, o_ref):
    o_ref[...] = jnp.maximum(_dot(x_ref[...], w_ref[...]) + b_ref[...], 0.0)


def _tc_in(x, w, b):
    return pl.pallas_call(
        _tc_in_body,
        grid=(GRID,),
        in_specs=[
            pl.BlockSpec((BLK, D), lambda i: (i, 0)),
            pl.BlockSpec((D, D), lambda i: (0, 0)),
            pl.BlockSpec((1, D), lambda i: (0, 0)),
        ],
        out_specs=pl.BlockSpec((BLK, D), lambda i: (i, 0)),
        out_shape=jax.ShapeDtypeStruct((N, D), jnp.float32),
    )(x, w, b)


def _tc_conv_body(parts_ref, h_ref, w_ref, b_ref, o_ref):
    a = parts_ref[0] + parts_ref[1] + h_ref[...]
    o_ref[...] = jnp.maximum(_dot(a, w_ref[...]) + b_ref[...], 0.0)


def _tc_conv(parts, h, w, b):
    return pl.pallas_call(
        _tc_conv_body,
        grid=(GRID,),
        in_specs=[
            pl.BlockSpec((NC, BLK, D), lambda i: (0, i, 0)),
            pl.BlockSpec((BLK, D), lambda i: (i, 0)),
            pl.BlockSpec((D, D), lambda i: (0, 0)),
            pl.BlockSpec((1, D), lambda i: (0, 0)),
        ],
        out_specs=pl.BlockSpec((BLK, D), lambda i: (i, 0)),
        out_shape=jax.ShapeDtypeStruct((N, D), jnp.float32),
    )(parts, h, w, b)


def _tc_final_body(parts_ref, h_ref, w_ref, b_ref, batch_ref,
                   wm_ref, bm_ref, wl_ref, bl_ref,
                   zm_ref, zl_ref, ge_acc):
    i = pl.program_id(0)
    a = parts_ref[0] + parts_ref[1] + h_ref[...]
    h3 = jnp.maximum(_dot(a, w_ref[...]) + b_ref[...], 0.0)

    batch_row = batch_ref[0, 0, :]
    gids = lax.broadcasted_iota(jnp.int32, (NUM_GRAPHS, BLK), 0)
    onehot = (batch_row[None, :] == gids).astype(jnp.float32)

    @pl.when(i == 0)
    def _():
        ge_acc[...] = jnp.zeros_like(ge_acc)

    ge_acc[...] += _dot(onehot, h3)

    @pl.when(i == GRID - 1)
    def _():
        ge = ge_acc[...]
        zm_ref[...] = _dot(ge, wm_ref[...]) + bm_ref[...]
        zl_ref[...] = _dot(ge, wl_ref[...]) + bl_ref[...]


def _tc_final(parts, h, w, b, batch_r, wm, bm, wl, bl):
    return pl.pallas_call(
        _tc_final_body,
        grid=(GRID,),
        in_specs=[
            pl.BlockSpec((NC, BLK, D), lambda i: (0, i, 0)),
            pl.BlockSpec((BLK, D), lambda i: (i, 0)),
            pl.BlockSpec((D, D), lambda i: (0, 0)),
            pl.BlockSpec((1, D), lambda i: (0, 0)),
            pl.BlockSpec((1, 1, BLK), lambda i: (i, 0, 0)),
            pl.BlockSpec((D, LATENT), lambda i: (0, 0)),
            pl.BlockSpec((1, LATENT), lambda i: (0, 0)),
            pl.BlockSpec((D, LATENT), lambda i: (0, 0)),
            pl.BlockSpec((1, LATENT), lambda i: (0, 0)),
        ],
        out_specs=[
            pl.BlockSpec((NUM_GRAPHS, LATENT), lambda i: (0, 0)),
            pl.BlockSpec((NUM_GRAPHS, LATENT), lambda i: (0, 0)),
        ],
        out_shape=[
            jax.ShapeDtypeStruct((NUM_GRAPHS, LATENT), jnp.float32),
            jax.ShapeDtypeStruct((NUM_GRAPHS, LATENT), jnp.float32),
        ],
        scratch_shapes=[pltpu.VMEM((NUM_GRAPHS, D), jnp.float32)],
    )(parts, h, w, b, batch_r, wm, bm, wl, bl)


def kernel(x, edge_index, batch, W_in, b_in, W0, b0, W1, b1, W2, b2,
           W_mean, b_mean, W_logvar, b_logvar):
    src = edge_index[0].astype(jnp.int32)
    dst = edge_index[1].astype(jnp.int32)
    pad = NW * PER_W - E
    trash = N + jnp.arange(pad, dtype=jnp.int32) % N_TRASH
    pad_src = jnp.arange(pad, dtype=jnp.int32) % N
    src_p = jnp.concatenate([src, pad_src]).reshape(NW, PER_W)
    dst_p = jnp.concatenate([dst, trash]).reshape(NW, PER_W)
    src_r = src_p.reshape(NW, NCHUNK, CH)
    dst_r = dst_p.reshape(NW, NCHUNK, CH)
    zeros_hbm = jnp.zeros((ROWS_PER_TILE, D), jnp.float32)
    batch_r = batch.astype(jnp.int32).reshape(GRID, 1, BLK)

    b_in2 = b_in.reshape(1, D)
    b02, b12, b22 = b0.reshape(1, D), b1.reshape(1, D), b2.reshape(1, D)
    bm2 = b_mean.reshape(1, LATENT)
    bl2 = b_logvar.reshape(1, LATENT)

    h = _tc_in(x, W_in, b_in2)
    for w, bb in ((W0, b02), (W1, b12)):
        parts = _sc_scatter(src_r, dst_r, h, zeros_hbm)
        h = _tc_conv(parts, h, w, bb)
    parts = _sc_scatter(src_r, dst_r, h, zeros_hbm)
    z_mean, z_logvar = _tc_final(parts, h, W2, b22, batch_r,
                                 W_mean, bm2, W_logvar, bl2)
    return (z_mean, z_logvar)
